# Initial kernel scaffold; baseline (speedup 1.0000x reference)
import jax
import jax.numpy as jnp
from jax.experimental import pallas as pl


def kernel(gammas, candidates, p_enc, q_enc, p_mask, q_mask):
    raise NotImplementedError("write your pallas kernel here")


# speedup vs baseline: 10.9448x; 10.9448x over previous
import functools

import jax
import jax.numpy as jnp
from jax import lax
from jax.experimental import pallas as pl
from jax.experimental.pallas import tpu as pltpu
from jax.experimental.pallas import tpu_sc as plsc


def kernel(gammas, candidates, p_enc, q_enc, p_mask, q_mask):
    B, P, T = gammas.shape
    NC = candidates.shape[1]
    d = T // 2
    R = B * NC * 2

    info = plsc.get_sparse_core_info()
    NCORE = info.num_cores
    NW = NCORE * info.num_subcores
    CH = 128
    rows_w = R // NW
    nch = rows_w // CH

    table = gammas.reshape(B * P * 2, d)
    cand2d = candidates.reshape(R // CH, CH)

    mesh = plsc.VectorSubcoreMesh(core_axis_name="c", subcore_axis_name="s")

    @functools.partial(
        pl.kernel,
        mesh=mesh,
        out_type=jax.ShapeDtypeStruct((R, d), jnp.float32),
        scratch_types=[
            pltpu.VMEM((nch, CH), jnp.int32),
            pltpu.VMEM((nch, CH), jnp.int32),
            pltpu.VMEM((CH, d), jnp.float32),
            pltpu.VMEM((CH, d), jnp.float32),
            pltpu.SemaphoreType.DMA,
            pltpu.SemaphoreType.DMA,
        ],
    )
    def sc_gather(table_hbm, cand_hbm, out_hbm, cand_v, idx_v, buf0, buf1,
                  sem0, sem1):
        wid = lax.axis_index("s") * NCORE + lax.axis_index("c")
        base = wid * rows_w
        pltpu.sync_copy(cand_hbm.at[pl.ds(wid * nch, nch)], cand_v)

        parity = lax.iota(jnp.int32, 16) & 1
        two_p = jnp.int32(2 * P)

        def build(i, carry):
            boff = ((base + i * CH) // (2 * NC)) * two_p
            for l in range(CH // 16):
                c = cand_v[i, pl.ds(l * 16, 16)]
                idx_v[i, pl.ds(l * 16, 16)] = c * 2 + parity + boff
            return carry

        lax.fori_loop(0, nch, build, 0)

        def step(t, carry):
            g0 = t * 2
            g1 = g0 + 1
            cp0 = pltpu.async_copy(table_hbm.at[idx_v.at[g0]], buf0, sem0)
            cp1 = pltpu.async_copy(table_hbm.at[idx_v.at[g1]], buf1, sem1)
            cp0.wait()
            pltpu.sync_copy(buf0, out_hbm.at[pl.ds(base + g0 * CH, CH)])
            cp1.wait()
            pltpu.sync_copy(buf1, out_hbm.at[pl.ds(base + g1 * CH, CH)])
            return carry

        lax.fori_loop(0, nch // 2, step, 0)

    out = sc_gather(table, cand2d)
    return out.reshape(B, NC, T)


# speedup vs baseline: 11.3907x; 1.0407x over previous
import functools

import jax
import jax.numpy as jnp
from jax import lax
from jax.experimental import pallas as pl
from jax.experimental.pallas import tpu as pltpu
from jax.experimental.pallas import tpu_sc as plsc


def kernel(gammas, candidates, p_enc, q_enc, p_mask, q_mask):
    B, P, T = gammas.shape
    NC = candidates.shape[1]
    d = T // 2
    R = B * NC * 2

    info = plsc.get_sparse_core_info()
    NCORE = info.num_cores
    NW = NCORE * info.num_subcores
    CH = 128
    rows_w = R // NW
    nch = rows_w // CH

    table = gammas.reshape(B * P * 2, d)
    cand2d = candidates.reshape(R // CH, CH)

    mesh = plsc.VectorSubcoreMesh(core_axis_name="c", subcore_axis_name="s")

    @functools.partial(
        pl.kernel,
        mesh=mesh,
        out_type=jax.ShapeDtypeStruct((R, d), jnp.float32),
        scratch_types=[
            pltpu.VMEM((nch, CH), jnp.int32),
            pltpu.VMEM((nch, CH), jnp.int32),
            pltpu.VMEM((CH, d), jnp.float32),
            pltpu.VMEM((CH, d), jnp.float32),
            pltpu.VMEM((CH, d), jnp.float32),
            pltpu.VMEM((CH, d), jnp.float32),
            pltpu.SemaphoreType.DMA,
            pltpu.SemaphoreType.DMA,
            pltpu.SemaphoreType.DMA,
            pltpu.SemaphoreType.DMA,
            pltpu.SemaphoreType.DMA,
            pltpu.SemaphoreType.DMA,
            pltpu.SemaphoreType.DMA,
            pltpu.SemaphoreType.DMA,
        ],
    )
    def sc_gather(table_hbm, cand_hbm, out_hbm, cand_v, idx_v,
                  buf0, buf1, buf2, buf3,
                  sg0, sg1, sg2, sg3, ss0, ss1, ss2, ss3):
        wid = lax.axis_index("s") * NCORE + lax.axis_index("c")
        base = wid * rows_w
        pltpu.sync_copy(cand_hbm.at[pl.ds(wid * nch, nch)], cand_v)

        parity = lax.iota(jnp.int32, 16) & 1
        two_p = jnp.int32(2 * P)

        def build(i, carry):
            boff = ((base + i * CH) // (2 * NC)) * two_p
            for l in range(CH // 16):
                c = cand_v[i, pl.ds(l * 16, 16)]
                idx_v[i, pl.ds(l * 16, 16)] = c * 2 + parity + boff
            return carry

        lax.fori_loop(0, nch, build, 0)

        bufs = (buf0, buf1, buf2, buf3)
        sg = (sg0, sg1, sg2, sg3)
        ss = (ss0, ss1, ss2, ss3)

        def wait_gather(g, k):
            pltpu.make_async_copy(table_hbm.at[idx_v.at[g]], bufs[k],
                                  sg[k]).wait()

        def wait_scatter(g, k):
            pltpu.make_async_copy(bufs[k],
                                  out_hbm.at[pl.ds(base + g * CH, CH)],
                                  ss[k]).wait()

        def step(t, carry):
            for k in range(4):
                g = 4 * t + k

                @pl.when(t > 0)
                def _():
                    wait_scatter(g - 4, k)

                pltpu.async_copy(table_hbm.at[idx_v.at[g]], bufs[k], sg[k])
                gm2 = g - 2
                km2 = (k + 2) % 4

                @pl.when(gm2 >= 0)
                def _():
                    wait_gather(gm2, km2)
                    pltpu.async_copy(
                        bufs[km2], out_hbm.at[pl.ds(base + gm2 * CH, CH)],
                        ss[km2])

            return carry

        lax.fori_loop(0, nch // 4, step, 0)

        for k in (2, 3):
            g = nch - 4 + k
            wait_gather(g, k)
            pltpu.async_copy(bufs[k], out_hbm.at[pl.ds(base + g * CH, CH)],
                             ss[k])
        for k in range(4):
            wait_scatter(nch - 4 + k, k)

    out = sc_gather(table, cand2d)
    return out.reshape(B, NC, T)


# speedup vs baseline: 33.5827x; 2.9483x over previous
import functools

import jax
import jax.numpy as jnp
from jax import lax
from jax.experimental import pallas as pl
from jax.experimental.pallas import tpu as pltpu
from jax.experimental.pallas import tpu_sc as plsc


def kernel(gammas, candidates, p_enc, q_enc, p_mask, q_mask):
    B, P, T = gammas.shape
    NC = candidates.shape[1]
    d = T // 2

    info = plsc.get_sparse_core_info()
    NCORE = info.num_cores
    NW = NCORE * info.num_subcores
    WPB = NW // B
    W = NC // WPB
    CH = 128
    nchk = W // CH
    nu = 2 * nchk

    mesh = plsc.VectorSubcoreMesh(core_axis_name="c", subcore_axis_name="s")

    @functools.partial(
        pl.kernel,
        mesh=mesh,
        out_type=jax.ShapeDtypeStruct((B, NC, T), jnp.float32),
        scratch_types=[
            pltpu.VMEM((nchk, CH), jnp.int32),
            pltpu.VMEM((nchk, CH), jnp.int32),
            pltpu.VMEM((4, CH, d), jnp.float32),
            pltpu.SemaphoreType.DMA,
            pltpu.SemaphoreType.DMA,
            pltpu.SemaphoreType.DMA,
            pltpu.SemaphoreType.DMA,
            pltpu.SemaphoreType.DMA,
            pltpu.SemaphoreType.DMA,
            pltpu.SemaphoreType.DMA,
            pltpu.SemaphoreType.DMA,
        ],
    )
    def sc_gather(gam_hbm, first_hbm, last_hbm, out_hbm, idxf_v, idxl_v,
                  bufs, sg0, sg1, sg2, sg3, ss0, ss1, ss2, ss3):
        wid = lax.axis_index("s") * NCORE + lax.axis_index("c")
        b = wid // WPB
        c0 = (wid % WPB) * W
        sg = (sg0, sg1, sg2, sg3)
        ss = (ss0, ss1, ss2, ss3)

        r0 = (wid % WPB) * nchk
        pltpu.sync_copy(first_hbm.at[b, pl.ds(r0, nchk)], idxf_v)
        pltpu.sync_copy(last_hbm.at[b, pl.ds(r0, nchk)], idxl_v)

        gam_f = gam_hbm.at[b, :, pl.ds(0, d)]
        gam_l = gam_hbm.at[b, :, pl.ds(d, d)]

        def gather_desc(g, k):
            if k % 2 == 0:
                return pltpu.make_async_copy(gam_f.at[idxf_v.at[g]],
                                             bufs.at[k], sg[k])
            return pltpu.make_async_copy(gam_l.at[idxl_v.at[g]],
                                         bufs.at[k], sg[k])

        def scatter_desc(g, k):
            cc = c0 + g * CH
            col = 0 if k % 2 == 0 else d
            return pltpu.make_async_copy(
                bufs.at[k], out_hbm.at[b, pl.ds(cc, CH), pl.ds(col, d)],
                ss[k])

        def step(t, carry):
            for k in range(4):
                u = 4 * t + k
                g = 2 * t + k // 2

                @pl.when(t > 0)
                def _():
                    scatter_desc(g - 2, k).wait()

                gather_desc(g, k).start()
                um2 = u - 2
                km2 = (k + 2) % 4
                gm2 = 2 * t + (k - 2) // 2

                @pl.when(um2 >= 0)
                def _():
                    gather_desc(gm2, km2).wait()
                    scatter_desc(gm2, km2).start()

            return carry

        lax.fori_loop(0, nu // 4, step, 0)

        for k in (2, 3):
            g = nchk - 1
            gather_desc(g, k).wait()
            scatter_desc(g, k).start()
        for k in range(4):
            scatter_desc(nchk - 2 + k // 2, k).wait()

    first_idx = candidates[:, :, 0].reshape(B, NC // CH, CH)
    last_idx = candidates[:, :, 1].reshape(B, NC // CH, CH)
    return sc_gather(gammas, first_idx, last_idx)


# speedup vs baseline: 34.6148x; 1.0307x over previous
import functools

import jax
import jax.numpy as jnp
from jax import lax
from jax.experimental import pallas as pl
from jax.experimental.pallas import tpu as pltpu
from jax.experimental.pallas import tpu_sc as plsc


def kernel(gammas, candidates, p_enc, q_enc, p_mask, q_mask):
    B, P, T = gammas.shape
    NC = candidates.shape[1]
    d = T // 2

    info = plsc.get_sparse_core_info()
    NCORE = info.num_cores
    NW = NCORE * info.num_subcores
    WPB = NW // B
    W = NC // WPB
    CH = 128
    nchk = W // CH

    mesh = plsc.VectorSubcoreMesh(core_axis_name="c", subcore_axis_name="s")

    @functools.partial(
        pl.kernel,
        mesh=mesh,
        out_type=jax.ShapeDtypeStruct((B, NC, T), jnp.float32),
        scratch_types=[
            pltpu.VMEM((nchk, CH), jnp.int32),
            pltpu.VMEM((nchk, CH), jnp.int32),
            pltpu.VMEM((3, CH, T), jnp.float32),
            pltpu.SemaphoreType.DMA,
            pltpu.SemaphoreType.DMA,
            pltpu.SemaphoreType.DMA,
            pltpu.SemaphoreType.DMA,
            pltpu.SemaphoreType.DMA,
            pltpu.SemaphoreType.DMA,
        ],
    )
    def sc_gather(gam_hbm, first_hbm, last_hbm, out_hbm, idxf_v, idxl_v,
                  bufs, sg0, sg1, sg2, ss0, ss1, ss2):
        wid = lax.axis_index("s") * NCORE + lax.axis_index("c")
        b = wid // WPB
        c0 = (wid % WPB) * W
        sg = (sg0, sg1, sg2)
        ss = (ss0, ss1, ss2)

        r0 = (wid % WPB) * nchk
        pltpu.sync_copy(first_hbm.at[b, pl.ds(r0, nchk)], idxf_v)
        pltpu.sync_copy(last_hbm.at[b, pl.ds(r0, nchk)], idxl_v)

        gam_f = gam_hbm.at[b, :, pl.ds(0, d)]
        gam_l = gam_hbm.at[b, :, pl.ds(d, d)]

        def gather_descs(g, k):
            return (
                pltpu.make_async_copy(gam_f.at[idxf_v.at[g]],
                                      bufs.at[k, :, pl.ds(0, d)], sg[k]),
                pltpu.make_async_copy(gam_l.at[idxl_v.at[g]],
                                      bufs.at[k, :, pl.ds(d, d)], sg[k]),
            )

        def issue_gathers(g, k):
            for c in gather_descs(g, k):
                c.start()

        def wait_gathers(g, k):
            for c in gather_descs(g, k):
                c.wait()

        def scatter_desc(g, k):
            return pltpu.make_async_copy(
                bufs.at[k], out_hbm.at[b, pl.ds(c0 + g * CH, CH)], ss[k])

        def step(t, carry):
            for k in range(3):
                g = 3 * t + k

                @pl.when(t > 0)
                def _():
                    scatter_desc(g - 3, k).wait()

                issue_gathers(g, k)
                gm1 = g - 1
                km1 = (k + 2) % 3

                @pl.when(gm1 >= 0)
                def _():
                    wait_gathers(gm1, km1)
                    scatter_desc(gm1, km1).start()

            return carry

        nfull = (nchk // 3) * 3
        lax.fori_loop(0, nchk // 3, step, 0)

        for g in range(nfull, nchk):
            k = g % 3
            scatter_desc(g - 3, k).wait()
            issue_gathers(g, k)
            wait_gathers(g - 1, (k + 2) % 3)
            scatter_desc(g - 1, (k + 2) % 3).start()

        wait_gathers(nchk - 1, (nchk - 1) % 3)
        scatter_desc(nchk - 1, (nchk - 1) % 3).start()
        for g in range(nchk - 3, nchk):
            scatter_desc(g, g % 3).wait()

    first_idx = candidates[:, :, 0].reshape(B, NC // CH, CH)
    last_idx = candidates[:, :, 1].reshape(B, NC // CH, CH)
    return sc_gather(gammas, first_idx, last_idx)


# speedup vs baseline: 34.9495x; 1.0097x over previous
import functools

import jax
import jax.numpy as jnp
from jax import lax
from jax.experimental import pallas as pl
from jax.experimental.pallas import tpu as pltpu
from jax.experimental.pallas import tpu_sc as plsc


def kernel(gammas, candidates, p_enc, q_enc, p_mask, q_mask):
    B, P, T = gammas.shape
    NC = candidates.shape[1]
    d = T // 2

    info = plsc.get_sparse_core_info()
    NCORE = info.num_cores
    NW = NCORE * info.num_subcores
    WPB = NW // B
    W = NC // WPB
    CH = 128
    nchk = W // CH

    mesh = plsc.VectorSubcoreMesh(core_axis_name="c", subcore_axis_name="s")

    @functools.partial(
        pl.kernel,
        mesh=mesh,
        out_type=jax.ShapeDtypeStruct((B, NC, T), jnp.float32),
        scratch_types=[
            pltpu.VMEM((nchk, CH), jnp.int32),
            pltpu.VMEM((nchk, CH), jnp.int32),
            pltpu.VMEM((3, CH, T), jnp.float32),
            pltpu.SemaphoreType.DMA,
            pltpu.SemaphoreType.DMA,
            pltpu.SemaphoreType.DMA,
            pltpu.SemaphoreType.DMA,
            pltpu.SemaphoreType.DMA,
            pltpu.SemaphoreType.DMA,
        ],
    )
    def sc_gather(gam_hbm, first_hbm, last_hbm, out_hbm, idxf_v, idxl_v,
                  bufs, sg0, sg1, sg2, ss0, ss1, ss2):
        wid = lax.axis_index("s") * NCORE + lax.axis_index("c")
        b = wid // WPB
        c0 = (wid % WPB) * W
        sg = (sg0, sg1, sg2)
        ss = (ss0, ss1, ss2)

        r0 = (wid % WPB) * nchk
        pltpu.sync_copy(first_hbm.at[b, pl.ds(r0, nchk)], idxf_v)
        pltpu.sync_copy(last_hbm.at[b, pl.ds(r0, nchk)], idxl_v)

        gam_f = gam_hbm.at[b, :, pl.ds(0, d)]
        gam_l = gam_hbm.at[b, :, pl.ds(d, d)]

        def gather_descs(g, k):
            return (
                pltpu.make_async_copy(gam_f.at[idxf_v.at[g]],
                                      bufs.at[k, :, pl.ds(0, d)], sg[k]),
                pltpu.make_async_copy(gam_l.at[idxl_v.at[g]],
                                      bufs.at[k, :, pl.ds(d, d)], sg[k]),
            )

        def issue_gathers(g, k):
            for c in gather_descs(g, k):
                c.start()

        def wait_gathers(g, k):
            for c in gather_descs(g, k):
                c.wait()

        def scatter_desc(g, k):
            return pltpu.make_async_copy(
                bufs.at[k], out_hbm.at[b, pl.ds(c0 + g * CH, CH)], ss[k])

        def step(t, carry):
            for k in range(3):
                g = 3 * t + k

                @pl.when(t > 0)
                def _():
                    scatter_desc(g - 3, k).wait()

                issue_gathers(g, k)
                gm2 = g - 2
                km2 = (k + 1) % 3

                @pl.when(gm2 >= 0)
                def _():
                    wait_gathers(gm2, km2)
                    scatter_desc(gm2, km2).start()

            return carry

        nfull = (nchk // 3) * 3
        lax.fori_loop(0, nchk // 3, step, 0)

        for g in range(nfull, nchk):
            k = g % 3
            scatter_desc(g - 3, k).wait()
            issue_gathers(g, k)
            wait_gathers(g - 2, (k + 1) % 3)
            scatter_desc(g - 2, (k + 1) % 3).start()

        for g in range(nchk - 2, nchk):
            wait_gathers(g, g % 3)
            scatter_desc(g, g % 3).start()
        for g in range(nchk - 3, nchk):
            scatter_desc(g, g % 3).wait()

    first_idx = candidates[:, :, 0].reshape(B, NC // CH, CH)
    last_idx = candidates[:, :, 1].reshape(B, NC // CH, CH)
    return sc_gather(gammas, first_idx, last_idx)
